# baseline (speedup 1.0000x reference)
import jax
import jax.numpy as jnp
from jax import lax
from jax.experimental import pallas as pl
from jax.experimental.pallas import tpu as pltpu
from jax.experimental.pallas import tpu_sc as plsc

D = 2048
QS = 4
DQ = D // QS
NCB = 32
L = 16
NC, NS = 2, 16
W = NC * NS
NPOS = 2 * 2048
PPW = NPOS // W
NBQ = (PPW // L) * QS
NT = PPW * QS
RB = 4


def _body(ids_hbm, offs_hbm, textq_hbm, audioq_hbm, out_hbm,
          ids_v, offs_v, aidx4, tidx4,
          ab0, ab1, ab2, ab3, ai0, ai1, ai2, ai3,
          tb0, tb1, ti0, ti1, ob0, ob1, oi0, oi1,
          as0, as1, as2, as3, ts0, ts1, os0, os1):
    abuf = [ab0, ab1, ab2, ab3]
    aidx_r = [ai0, ai1, ai2, ai3]
    tbuf = [tb0, tb1]
    tidx_r = [ti0, ti1]
    obuf = [ob0, ob1]
    oidx_r = [oi0, oi1]
    asem = [as0, as1, as2, as3]
    tsem = [ts0, ts1]
    osem = [os0, os1]

    wid = lax.axis_index("s") * NC + lax.axis_index("c")
    pbase = wid * PPW
    iot = lax.iota(jnp.int32, L)

    pltpu.sync_copy(ids_hbm.at[pl.ds(pbase * (NCB + 1), PPW * (NCB + 1))], ids_v)
    pltpu.sync_copy(offs_hbm, offs_v)
    o0 = offs_v[pl.ds(0, L)]
    o1 = offs_v[pl.ds(L, L)]

    def mk_aidx(p, c):
        a0 = ids_v[pl.ds(p * (NCB + 1), L)]
        a1 = ids_v[pl.ds(p * (NCB + 1) + L, L)]
        aidx4[pl.ds(p * NCB, L)] = jnp.where(a0 != 0, (a0 + o0) * QS, 0)
        aidx4[pl.ds(p * NCB + L, L)] = jnp.where(a1 != 0, (a1 + o1) * QS, 0)
        a2 = ids_v[pl.ds(p * (NCB + 1) + NCB - (L - 1), L)]
        tid = a2[L - 1] * QS
        base = (p >> 4) * L
        tv = tidx4[pl.ds(base, L)]
        tidx4[pl.ds(base, L)] = jnp.where(iot == (p & (L - 1)), tid, tv)
        return c
    lax.fori_loop(0, PPW, mk_aidx, 0)

    def issue_audio(t, r):
        bq = t >> 4
        q = bq & 3
        lp = (bq >> 2) * L + (t & 15)
        aidx_r[r][pl.ds(0, L)] = aidx4[pl.ds(lp * NCB, L)] + q
        aidx_r[r][pl.ds(L, L)] = aidx4[pl.ds(lp * NCB + L, L)] + q
        pltpu.async_copy(audioq_hbm.at[aidx_r[r]], abuf[r], asem[r])

    def wait_audio(r):
        pltpu.make_async_copy(audioq_hbm.at[aidx_r[r]], abuf[r], asem[r]).wait()

    def issue_text(bq, par):
        q = bq & 3
        tidx_r[par][pl.ds(0, L)] = tidx4[pl.ds((bq >> 2) * L, L)] + q
        pltpu.async_copy(textq_hbm.at[tidx_r[par]], tbuf[par], tsem[par])

    def wait_text(par):
        pltpu.make_async_copy(textq_hbm.at[tidx_r[par]], tbuf[par], tsem[par]).wait()

    def issue_scatter(bq, par):
        q = bq & 3
        oidx_r[par][pl.ds(0, L)] = (pbase + (bq >> 2) * L + iot) * QS + q
        pltpu.async_copy(obuf[par], out_hbm.at[oidx_r[par]], osem[par])

    def wait_scatter(par):
        pltpu.make_async_copy(obuf[par], out_hbm.at[oidx_r[par]], osem[par]).wait()

    def accumulate(t, r, par):
        p = t & 15
        def d_body(d, c):
            col = d * L
            acc = tbuf[par][p, pl.ds(col, L)]
            for cb in range(NCB):
                acc = acc + abuf[r][cb, pl.ds(col, L)]
            obuf[par][p, pl.ds(col, L)] = acc
            return c
        lax.fori_loop(0, DQ // L, d_body, 0)

    issue_text(0, 0)
    issue_text(1, 1)
    for r in range(RB):
        issue_audio(r, r)

    def bq2_body(bq2, c):
        for par in range(2):
            bq = bq2 * 2 + par
            wait_text(par)

            @pl.when(bq >= 2)
            def _():
                wait_scatter(par)

            def g_body(g, cc):
                t0 = bq * L + g * RB
                for r in range(RB):
                    t = t0 + r
                    wait_audio(r)
                    accumulate(t, r, par)

                    @pl.when(t + RB < NT)
                    def _():
                        issue_audio(t + RB, r)
                return cc
            lax.fori_loop(0, L // RB, g_body, 0)

            issue_scatter(bq, par)

            @pl.when(bq + 2 < NBQ)
            def _():
                issue_text(bq + 2, par)
        return c
    lax.fori_loop(0, NBQ // 2, bq2_body, 0)
    wait_scatter(0)
    wait_scatter(1)


def kernel(input_ids, text_table, audio_table, audio_tokens_offsets):
    ids = input_ids.astype(jnp.int32).reshape(-1)
    offs = audio_tokens_offsets.astype(jnp.int32)
    textq = text_table.reshape(-1, DQ)
    audioq = audio_table.reshape(-1, DQ)

    mesh = plsc.VectorSubcoreMesh(
        core_axis_name="c", subcore_axis_name="s",
        num_cores=NC, num_subcores=NS)
    scratch = [
        pltpu.VMEM((PPW * (NCB + 1),), jnp.int32),
        pltpu.VMEM((NCB,), jnp.int32),
        pltpu.VMEM((PPW * NCB,), jnp.int32),
        pltpu.VMEM((PPW,), jnp.int32),
    ]
    scratch += [pltpu.VMEM((NCB, DQ), jnp.float32) for _ in range(RB)]
    scratch += [pltpu.VMEM((NCB,), jnp.int32) for _ in range(RB)]
    scratch += [pltpu.VMEM((L, DQ), jnp.float32) for _ in range(2)]
    scratch += [pltpu.VMEM((L,), jnp.int32) for _ in range(2)]
    scratch += [pltpu.VMEM((L, DQ), jnp.float32) for _ in range(2)]
    scratch += [pltpu.VMEM((L,), jnp.int32) for _ in range(2)]
    scratch += [pltpu.SemaphoreType.DMA for _ in range(8)]

    run = pl.kernel(
        _body,
        out_type=jax.ShapeDtypeStruct((NPOS * QS, DQ), jnp.float32),
        mesh=mesh,
        scratch_types=scratch,
    )
    out = run(ids, offs, textq, audioq)
    return out.reshape(input_ids.shape[0], input_ids.shape[1], D)


# speedup vs baseline: 3.3866x; 3.3866x over previous
import jax
import jax.numpy as jnp
from jax import lax
from jax.experimental import pallas as pl
from jax.experimental.pallas import tpu as pltpu
from jax.experimental.pallas import tpu_sc as plsc

D = 2048
QS = 4
DQ = D // QS
NCB = 32
L = 16
NC, NS = 2, 16
W = NC * NS
NPOS = 2 * 2048
PPW = NPOS // W
NBQ = (PPW // L) * QS
NT = PPW * QS
RB = 4


def _body(ids_hbm, offs_hbm, textq_hbm, audioq_hbm, out_hbm,
          ids_v, offs_v, aidx4, tidx4,
          ab0, ab1, ab2, ab3, ai0, ai1, ai2, ai3,
          tb0, tb1, ti0, ti1, ob0, ob1, oi0, oi1,
          as0, as1, as2, as3, ts0, ts1, os0, os1):
    abuf = [ab0, ab1, ab2, ab3]
    aidx_r = [ai0, ai1, ai2, ai3]
    tbuf = [tb0, tb1]
    tidx_r = [ti0, ti1]
    obuf = [ob0, ob1]
    oidx_r = [oi0, oi1]
    asem = [as0, as1, as2, as3]
    tsem = [ts0, ts1]
    osem = [os0, os1]

    wid = lax.axis_index("s") * NC + lax.axis_index("c")
    pbase = wid * PPW
    iot = lax.iota(jnp.int32, L)

    pltpu.sync_copy(ids_hbm.at[pl.ds(pbase * (NCB + 1), PPW * (NCB + 1))], ids_v)
    pltpu.sync_copy(offs_hbm, offs_v)
    o0 = offs_v[pl.ds(0, L)]
    o1 = offs_v[pl.ds(L, L)]

    def mk_aidx(p, c):
        a0 = ids_v[pl.ds(p * (NCB + 1), L)]
        a1 = ids_v[pl.ds(p * (NCB + 1) + L, L)]
        aidx4[pl.ds(p * NCB, L)] = jnp.where(a0 != 0, a0 + o0, 0)
        aidx4[pl.ds(p * NCB + L, L)] = jnp.where(a1 != 0, a1 + o1, 0)
        a2 = ids_v[pl.ds(p * (NCB + 1) + NCB - (L - 1), L)]
        tid = a2[L - 1]
        base = (p >> 4) * L
        tv = tidx4[pl.ds(base, L)]
        tidx4[pl.ds(base, L)] = jnp.where(iot == (p & (L - 1)), tid, tv)
        return c
    lax.fori_loop(0, PPW, mk_aidx, 0)

    def issue_audio(t, r):
        bq = t >> 4
        q = bq & 3
        lp = (bq >> 2) * L + (t & 15)
        aidx_r[r][pl.ds(0, L)] = aidx4[pl.ds(lp * NCB, L)]
        aidx_r[r][pl.ds(L, L)] = aidx4[pl.ds(lp * NCB + L, L)]
        pltpu.async_copy(audioq_hbm.at[aidx_r[r], pl.ds(q * DQ, DQ)], abuf[r], asem[r])

    def wait_audio(r):
        pltpu.make_async_copy(audioq_hbm.at[aidx_r[r], pl.ds(0, DQ)], abuf[r], asem[r]).wait()

    def issue_text(bq, par):
        q = bq & 3
        tidx_r[par][pl.ds(0, L)] = tidx4[pl.ds((bq >> 2) * L, L)]
        pltpu.async_copy(textq_hbm.at[tidx_r[par], pl.ds(q * DQ, DQ)], tbuf[par], tsem[par])

    def wait_text(par):
        pltpu.make_async_copy(textq_hbm.at[tidx_r[par], pl.ds(0, DQ)], tbuf[par], tsem[par]).wait()

    def issue_scatter(bq, par):
        q = bq & 3
        oidx_r[par][pl.ds(0, L)] = pbase + (bq >> 2) * L + iot
        pltpu.async_copy(obuf[par], out_hbm.at[oidx_r[par], pl.ds(q * DQ, DQ)], osem[par])

    def wait_scatter(par):
        pltpu.make_async_copy(obuf[par], out_hbm.at[oidx_r[par], pl.ds(0, DQ)], osem[par]).wait()

    def accumulate(t, r, par):
        p = t & 15
        def d_body(d, c):
            col = d * L
            acc = tbuf[par][p, pl.ds(col, L)]
            for cb in range(NCB):
                acc = acc + abuf[r][cb, pl.ds(col, L)]
            obuf[par][p, pl.ds(col, L)] = acc
            return c
        lax.fori_loop(0, DQ // L, d_body, 0)

    issue_text(0, 0)
    issue_text(1, 1)
    for r in range(RB):
        issue_audio(r, r)

    def bq2_body(bq2, c):
        for par in range(2):
            bq = bq2 * 2 + par
            wait_text(par)

            @pl.when(bq >= 2)
            def _():
                wait_scatter(par)

            def g_body(g, cc):
                t0 = bq * L + g * RB
                for r in range(RB):
                    t = t0 + r
                    wait_audio(r)
                    accumulate(t, r, par)

                    @pl.when(t + RB < NT)
                    def _():
                        issue_audio(t + RB, r)
                return cc
            lax.fori_loop(0, L // RB, g_body, 0)

            issue_scatter(bq, par)

            @pl.when(bq + 2 < NBQ)
            def _():
                issue_text(bq + 2, par)
        return c
    lax.fori_loop(0, NBQ // 2, bq2_body, 0)
    wait_scatter(0)
    wait_scatter(1)


def kernel(input_ids, text_table, audio_table, audio_tokens_offsets):
    ids = input_ids.astype(jnp.int32).reshape(-1)
    offs = audio_tokens_offsets.astype(jnp.int32)

    mesh = plsc.VectorSubcoreMesh(
        core_axis_name="c", subcore_axis_name="s",
        num_cores=NC, num_subcores=NS)
    scratch = [
        pltpu.VMEM((PPW * (NCB + 1),), jnp.int32),
        pltpu.VMEM((NCB,), jnp.int32),
        pltpu.VMEM((PPW * NCB,), jnp.int32),
        pltpu.VMEM((PPW,), jnp.int32),
    ]
    scratch += [pltpu.VMEM((NCB, DQ), jnp.float32) for _ in range(RB)]
    scratch += [pltpu.VMEM((NCB,), jnp.int32) for _ in range(RB)]
    scratch += [pltpu.VMEM((L, DQ), jnp.float32) for _ in range(2)]
    scratch += [pltpu.VMEM((L,), jnp.int32) for _ in range(2)]
    scratch += [pltpu.VMEM((L, DQ), jnp.float32) for _ in range(2)]
    scratch += [pltpu.VMEM((L,), jnp.int32) for _ in range(2)]
    scratch += [pltpu.SemaphoreType.DMA for _ in range(8)]

    run = pl.kernel(
        _body,
        out_type=jax.ShapeDtypeStruct((NPOS, D), jnp.float32),
        mesh=mesh,
        scratch_types=scratch,
    )
    out = run(ids, offs, text_table, audio_table)
    return out.reshape(input_ids.shape[0], input_ids.shape[1], D)


# speedup vs baseline: 4.6874x; 1.3841x over previous
import jax
import jax.numpy as jnp
from jax import lax
from jax.experimental import pallas as pl
from jax.experimental.pallas import tpu as pltpu
from jax.experimental.pallas import tpu_sc as plsc

D = 2048
QS = 4
DQ = D // QS
NCB = 32
L = 16
NC, NS = 2, 16
W = NC * NS
NPOS = 2 * 2048
PPW = NPOS // W
NBQ = (PPW // L) * QS
NT = PPW * QS
RB = 4


def _body(ids_hbm, offs_hbm, textq_hbm, audioq_hbm, out_hbm,
          ids_v, offs_v, aidx4, tidx4,
          ab0, ab1, ab2, ab3, ai0, ai1, ai2, ai3,
          tb0, tb1, ti0, ti1, ob0, ob1, oi0, oi1,
          as0, as1, as2, as3, ts0, ts1, os0, os1):
    abuf = [ab0, ab1, ab2, ab3]
    aidx_r = [ai0, ai1, ai2, ai3]
    tbuf = [tb0, tb1]
    tidx_r = [ti0, ti1]
    obuf = [ob0, ob1]
    oidx_r = [oi0, oi1]
    asem = [as0, as1, as2, as3]
    tsem = [ts0, ts1]
    osem = [os0, os1]

    wid = lax.axis_index("s") * NC + lax.axis_index("c")
    pbase = wid * PPW
    iot = lax.iota(jnp.int32, L)

    pltpu.sync_copy(ids_hbm.at[pl.ds(pbase * (NCB + 1), PPW * (NCB + 1))], ids_v)
    pltpu.sync_copy(offs_hbm, offs_v)
    o0 = offs_v[pl.ds(0, L)]
    o1 = offs_v[pl.ds(L, L)]

    def mk_aidx(p, c):
        a0 = ids_v[pl.ds(p * (NCB + 1), L)]
        a1 = ids_v[pl.ds(p * (NCB + 1) + L, L)]
        aidx4[pl.ds(p * NCB, L)] = jnp.where(a0 != 0, a0 + o0, 0)
        aidx4[pl.ds(p * NCB + L, L)] = jnp.where(a1 != 0, a1 + o1, 0)
        a2 = ids_v[pl.ds(p * (NCB + 1) + NCB - (L - 1), L)]
        tid = a2[L - 1]
        base = (p >> 4) * L
        tv = tidx4[pl.ds(base, L)]
        tidx4[pl.ds(base, L)] = jnp.where(iot == (p & (L - 1)), tid, tv)
        return c
    lax.fori_loop(0, PPW, mk_aidx, 0)

    def issue_audio(t, r):
        bq = t >> 4
        q = bq & 3
        lp = (bq >> 2) * L + (t & 15)
        aidx_r[r][pl.ds(0, L)] = aidx4[pl.ds(lp * NCB, L)]
        aidx_r[r][pl.ds(L, L)] = aidx4[pl.ds(lp * NCB + L, L)]
        pltpu.async_copy(audioq_hbm.at[aidx_r[r], pl.ds(q * DQ, DQ)], abuf[r], asem[r])

    def wait_audio(r):
        pltpu.make_async_copy(audioq_hbm.at[aidx_r[r], pl.ds(0, DQ)], abuf[r], asem[r]).wait()

    def issue_text(bq, par):
        q = bq & 3
        tidx_r[par][pl.ds(0, L)] = tidx4[pl.ds((bq >> 2) * L, L)]
        pltpu.async_copy(textq_hbm.at[tidx_r[par], pl.ds(q * DQ, DQ)], tbuf[par], tsem[par])

    def wait_text(par):
        pltpu.make_async_copy(textq_hbm.at[tidx_r[par], pl.ds(0, DQ)], tbuf[par], tsem[par]).wait()

    def issue_scatter(bq, par):
        q = bq & 3
        oidx_r[par][pl.ds(0, L)] = pbase + (bq >> 2) * L + iot
        pltpu.async_copy(obuf[par], out_hbm.at[oidx_r[par], pl.ds(q * DQ, DQ)], osem[par])

    def wait_scatter(par):
        pltpu.make_async_copy(obuf[par], out_hbm.at[oidx_r[par], pl.ds(0, DQ)], osem[par]).wait()

    def accumulate(t, r, par):
        p = t & 15
        def d_body(d, c):
            for u in range(2):
                col = d * (2 * L) + u * L
                vals = [tbuf[par][p, pl.ds(col, L)]]
                vals += [abuf[r][cb, pl.ds(col, L)] for cb in range(NCB)]
                while len(vals) > 1:
                    nxt = [vals[i] + vals[i + 1] for i in range(0, len(vals) - 1, 2)]
                    if len(vals) % 2:
                        nxt.append(vals[-1])
                    vals = nxt
                obuf[par][p, pl.ds(col, L)] = vals[0]
            return c
        lax.fori_loop(0, DQ // (2 * L), d_body, 0)

    issue_text(0, 0)
    issue_text(1, 1)
    for r in range(RB):
        issue_audio(r, r)

    def bq2_body(bq2, c):
        for par in range(2):
            bq = bq2 * 2 + par
            wait_text(par)

            @pl.when(bq >= 2)
            def _():
                wait_scatter(par)

            def g_body(g, cc):
                t0 = bq * L + g * RB
                for r in range(RB):
                    t = t0 + r
                    wait_audio(r)
                    accumulate(t, r, par)

                    @pl.when(t + RB < NT)
                    def _():
                        issue_audio(t + RB, r)
                return cc
            lax.fori_loop(0, L // RB, g_body, 0)

            issue_scatter(bq, par)

            @pl.when(bq + 2 < NBQ)
            def _():
                issue_text(bq + 2, par)
        return c
    lax.fori_loop(0, NBQ // 2, bq2_body, 0)
    wait_scatter(0)
    wait_scatter(1)


def kernel(input_ids, text_table, audio_table, audio_tokens_offsets):
    ids = input_ids.astype(jnp.int32).reshape(-1)
    offs = audio_tokens_offsets.astype(jnp.int32)

    mesh = plsc.VectorSubcoreMesh(
        core_axis_name="c", subcore_axis_name="s",
        num_cores=NC, num_subcores=NS)
    scratch = [
        pltpu.VMEM((PPW * (NCB + 1),), jnp.int32),
        pltpu.VMEM((NCB,), jnp.int32),
        pltpu.VMEM((PPW * NCB,), jnp.int32),
        pltpu.VMEM((PPW,), jnp.int32),
    ]
    scratch += [pltpu.VMEM((NCB, DQ), jnp.float32) for _ in range(RB)]
    scratch += [pltpu.VMEM((NCB,), jnp.int32) for _ in range(RB)]
    scratch += [pltpu.VMEM((L, DQ), jnp.float32) for _ in range(2)]
    scratch += [pltpu.VMEM((L,), jnp.int32) for _ in range(2)]
    scratch += [pltpu.VMEM((L, DQ), jnp.float32) for _ in range(2)]
    scratch += [pltpu.VMEM((L,), jnp.int32) for _ in range(2)]
    scratch += [pltpu.SemaphoreType.DMA for _ in range(8)]

    run = pl.kernel(
        _body,
        out_type=jax.ShapeDtypeStruct((NPOS, D), jnp.float32),
        mesh=mesh,
        scratch_types=scratch,
    )
    out = run(ids, offs, text_table, audio_table)
    return out.reshape(input_ids.shape[0], input_ids.shape[1], D)
